# initial kernel scaffold (unmeasured)
import jax
import jax.numpy as jnp
from jax import lax
from jax.experimental import pallas as pl
from jax.experimental.pallas import tpu as pltpu

N_DEV = 32


def kernel(x, w_mat):
    m_tot, k_shard = x.shape
    _, n = w_mat.shape
    m_per = m_tot // N_DEV

    def body(x_ref, w_ref, out_ref, wbf_ref, comm_ref, send_sems, recv_sems):
        my = lax.axis_index("i")
        left = lax.rem(my - 1 + N_DEV, N_DEV)
        right = lax.rem(my + 1, N_DEV)

        barrier_sem = pltpu.get_barrier_semaphore()
        for nbr in (left, right):
            pl.semaphore_signal(
                barrier_sem, inc=1,
                device_id=(nbr,), device_id_type=pl.DeviceIdType.MESH,
            )
        pl.semaphore_wait(barrier_sem, 2)

        wbf_ref[...] = w_ref[...].astype(jnp.bfloat16)

        def partial_for(c):
            xb = x_ref[pl.ds(c * m_per, m_per), :].astype(jnp.bfloat16)
            return jnp.dot(xb, wbf_ref[...], preferred_element_type=jnp.float32)

        c0 = lax.rem(my - 1 + N_DEV, N_DEV)
        comm_ref[N_DEV - 1] = partial_for(c0).astype(jnp.bfloat16)

        for h in range(N_DEV - 1):
            src_slot = N_DEV - 1 if h == 0 else h - 1
            rdma = pltpu.make_async_remote_copy(
                src_ref=comm_ref.at[src_slot],
                dst_ref=comm_ref.at[h],
                send_sem=send_sems.at[h],
                recv_sem=recv_sems.at[h],
                device_id=(right,),
                device_id_type=pl.DeviceIdType.MESH,
            )
            rdma.start()
            rdma.wait()

            c = lax.rem(my - h - 2 + 2 * N_DEV, N_DEV)
            acc = comm_ref[h].astype(jnp.float32) + partial_for(c)
            if h < N_DEV - 2:
                comm_ref[h] = acc.astype(jnp.bfloat16)
            else:
                out_ref[...] = acc * jax.nn.sigmoid(acc)

    return pl.pallas_call(
        body,
        out_shape=jax.ShapeDtypeStruct((m_per, n), jnp.float32),
        in_specs=[
            pl.BlockSpec(memory_space=pltpu.VMEM),
            pl.BlockSpec(memory_space=pltpu.VMEM),
        ],
        out_specs=pl.BlockSpec(memory_space=pltpu.VMEM),
        scratch_shapes=[
            pltpu.VMEM((m_per, n), jnp.bfloat16),
            pltpu.VMEM((N_DEV, m_per, n), jnp.bfloat16),
            pltpu.SemaphoreType.DMA((N_DEV - 1,)),
            pltpu.SemaphoreType.DMA((N_DEV - 1,)),
        ],
        compiler_params=pltpu.CompilerParams(collective_id=0),
    )(x, w_mat)


# baseline (device time: 249650 ns/iter reference)
import jax
import jax.numpy as jnp
from jax import lax
from jax.experimental import pallas as pl
from jax.experimental.pallas import tpu as pltpu

N_DEV = 32


def kernel(x, w_mat):
    m_tot, k_shard = x.shape
    _, n = w_mat.shape
    m_per = m_tot // N_DEV

    def body(x_ref, w_ref, out_ref, wbf_ref, comm_ref, send_sems, recv_sems):
        my = lax.axis_index("i")
        left = lax.rem(my - 1 + N_DEV, N_DEV)
        right = lax.rem(my + 1, N_DEV)

        barrier_sem = pltpu.get_barrier_semaphore()
        for nbr in (left, right):
            pl.semaphore_signal(
                barrier_sem, inc=1,
                device_id=(nbr,), device_id_type=pl.DeviceIdType.MESH,
            )
        pl.semaphore_wait(barrier_sem, 2)

        wbf_ref[...] = w_ref[...].astype(jnp.bfloat16)

        def partial_for(c):
            xb = x_ref[pl.ds(c * m_per, m_per), :].astype(jnp.bfloat16)
            return jnp.dot(xb, wbf_ref[...], preferred_element_type=jnp.float32)

        c0 = lax.rem(my - 1 + N_DEV, N_DEV)
        comm_ref[N_DEV - 1] = partial_for(c0).astype(jnp.bfloat16)

        for h in range(N_DEV - 1):
            src_slot = N_DEV - 1 if h == 0 else h - 1
            rdma = pltpu.make_async_remote_copy(
                src_ref=comm_ref.at[src_slot],
                dst_ref=comm_ref.at[h],
                send_sem=send_sems.at[h],
                recv_sem=recv_sems.at[h],
                device_id=(right,),
                device_id_type=pl.DeviceIdType.MESH,
            )
            rdma.start()
            rdma.wait()

            c = lax.rem(my - h - 2 + 2 * N_DEV, N_DEV)
            acc = comm_ref[h].astype(jnp.float32) + partial_for(c)
            if h < N_DEV - 2:
                comm_ref[h] = acc.astype(jnp.bfloat16)
            else:
                out_ref[...] = acc * jax.nn.sigmoid(acc)

    return pl.pallas_call(
        body,
        out_shape=jax.ShapeDtypeStruct((m_per, n), jnp.float32),
        in_specs=[
            pl.BlockSpec(memory_space=pltpu.VMEM),
            pl.BlockSpec(memory_space=pltpu.VMEM),
        ],
        out_specs=pl.BlockSpec(memory_space=pltpu.VMEM),
        scratch_shapes=[
            pltpu.VMEM((k_shard, n), jnp.bfloat16),
            pltpu.VMEM((N_DEV, m_per, n), jnp.bfloat16),
            pltpu.SemaphoreType.DMA((N_DEV - 1,)),
            pltpu.SemaphoreType.DMA((N_DEV - 1,)),
        ],
        compiler_params=pltpu.CompilerParams(collective_id=0),
    )(x, w_mat)


# device time: 191110 ns/iter; 1.3063x vs baseline; 1.3063x over previous
import jax
import jax.numpy as jnp
from jax import lax
from jax.experimental import pallas as pl
from jax.experimental.pallas import tpu as pltpu

N_DEV = 32
SUB = 1


def kernel(x, w_mat):
    m_tot, k_shard = x.shape
    _, n = w_mat.shape
    m_per = m_tot // N_DEV
    nh = n // 2
    ns = nh // SUB

    def body(x_ref, w_ref, out_ref, wbf_ref, cw_ref, ccw_ref,
             cw_send, cw_recv, ccw_send, ccw_recv):
        my = lax.axis_index("i")
        left = lax.rem(my - 1 + N_DEV, N_DEV)
        right = lax.rem(my + 1, N_DEV)

        barrier_sem = pltpu.get_barrier_semaphore()
        for nbr in (left, right):
            pl.semaphore_signal(
                barrier_sem, inc=1,
                device_id=(nbr,), device_id_type=pl.DeviceIdType.MESH,
            )
        pl.semaphore_wait(barrier_sem, 2)

        wbf_ref[...] = w_ref[...].astype(jnp.bfloat16)

        def partial_for(c, col0):
            xb = x_ref[pl.ds(c * m_per, m_per), :].astype(jnp.bfloat16)
            return jnp.dot(xb, wbf_ref[:, col0:col0 + nh],
                           preferred_element_type=jnp.float32)

        def mk(dir_ref, send_sems, recv_sems, dst_dev, h, s):
            src_slot = N_DEV - 1 if h == 0 else h - 1
            return pltpu.make_async_remote_copy(
                src_ref=dir_ref.at[src_slot, :, s * ns:(s + 1) * ns],
                dst_ref=dir_ref.at[h, :, s * ns:(s + 1) * ns],
                send_sem=send_sems.at[h, s],
                recv_sem=recv_sems.at[h, s],
                device_id=(dst_dev,),
                device_id_type=pl.DeviceIdType.MESH,
            )

        cw_ref[N_DEV - 1] = partial_for(
            lax.rem(my - 1 + N_DEV, N_DEV), 0).astype(jnp.bfloat16)
        ccw_ref[N_DEV - 1] = partial_for(
            lax.rem(my + 1, N_DEV), nh).astype(jnp.bfloat16)
        for s in range(SUB):
            mk(cw_ref, cw_send, cw_recv, right, 0, s).start()
            mk(ccw_ref, ccw_send, ccw_recv, left, 0, s).start()

        for h in range(N_DEV - 1):
            c_cw = lax.rem(my - h - 2 + 2 * N_DEV, N_DEV)
            c_ccw = lax.rem(my + h + 2, N_DEV)
            p_cw = partial_for(c_cw, 0)
            p_ccw = partial_for(c_ccw, nh)

            for s in range(SUB):
                sl = slice(s * ns, (s + 1) * ns)
                for dir_ref, send_sems, recv_sems, dst, p, col0 in (
                    (cw_ref, cw_send, cw_recv, right, p_cw, 0),
                    (ccw_ref, ccw_send, ccw_recv, left, p_ccw, nh),
                ):
                    mk(dir_ref, send_sems, recv_sems, dst, h, s).wait_recv()
                    acc = dir_ref[h, :, sl].astype(jnp.float32) + p[:, sl]
                    if h < N_DEV - 2:
                        dir_ref[h, :, sl] = acc.astype(jnp.bfloat16)
                        mk(dir_ref, send_sems, recv_sems, dst, h + 1, s).start()
                    else:
                        out_ref[:, col0 + s * ns:col0 + (s + 1) * ns] = (
                            acc * jax.nn.sigmoid(acc))

        for h in range(N_DEV - 1):
            for s in range(SUB):
                mk(cw_ref, cw_send, cw_recv, right, h, s).wait_send()
                mk(ccw_ref, ccw_send, ccw_recv, left, h, s).wait_send()

    return pl.pallas_call(
        body,
        out_shape=jax.ShapeDtypeStruct((m_per, n), jnp.float32),
        in_specs=[
            pl.BlockSpec(memory_space=pltpu.VMEM),
            pl.BlockSpec(memory_space=pltpu.VMEM),
        ],
        out_specs=pl.BlockSpec(memory_space=pltpu.VMEM),
        scratch_shapes=[
            pltpu.VMEM((k_shard, n), jnp.bfloat16),
            pltpu.VMEM((N_DEV, m_per, nh), jnp.bfloat16),
            pltpu.VMEM((N_DEV, m_per, nh), jnp.bfloat16),
            pltpu.SemaphoreType.DMA((N_DEV - 1, SUB)),
            pltpu.SemaphoreType.DMA((N_DEV - 1, SUB)),
            pltpu.SemaphoreType.DMA((N_DEV - 1, SUB)),
            pltpu.SemaphoreType.DMA((N_DEV - 1, SUB)),
        ],
        compiler_params=pltpu.CompilerParams(collective_id=0),
    )(x, w_mat)


# device time: 187495 ns/iter; 1.3315x vs baseline; 1.0193x over previous
import jax
import jax.numpy as jnp
from jax import lax
from jax.experimental import pallas as pl
from jax.experimental.pallas import tpu as pltpu

N_DEV = 32
SUB = 2


def kernel(x, w_mat):
    m_tot, k_shard = x.shape
    _, n = w_mat.shape
    m_per = m_tot // N_DEV
    nh = n // 2
    ns = nh // SUB

    def body(x_ref, w_ref, out_ref, wbf_ref, cw_ref, ccw_ref,
             cw_send, cw_recv, ccw_send, ccw_recv):
        my = lax.axis_index("i")
        left = lax.rem(my - 1 + N_DEV, N_DEV)
        right = lax.rem(my + 1, N_DEV)

        barrier_sem = pltpu.get_barrier_semaphore()
        for nbr in (left, right):
            pl.semaphore_signal(
                barrier_sem, inc=1,
                device_id=(nbr,), device_id_type=pl.DeviceIdType.MESH,
            )
        pl.semaphore_wait(barrier_sem, 2)

        wbf_ref[...] = w_ref[...].astype(jnp.bfloat16)

        def partial_for(c, col0):
            xb = x_ref[pl.ds(c * m_per, m_per), :].astype(jnp.bfloat16)
            return jnp.dot(xb, wbf_ref[:, col0:col0 + nh],
                           preferred_element_type=jnp.float32)

        def mk(dir_ref, send_sems, recv_sems, dst_dev, h, s):
            src_slot = N_DEV - 1 if h == 0 else h - 1
            return pltpu.make_async_remote_copy(
                src_ref=dir_ref.at[src_slot, :, s * ns:(s + 1) * ns],
                dst_ref=dir_ref.at[h, :, s * ns:(s + 1) * ns],
                send_sem=send_sems.at[h, s],
                recv_sem=recv_sems.at[h, s],
                device_id=(dst_dev,),
                device_id_type=pl.DeviceIdType.MESH,
            )

        cw_ref[N_DEV - 1] = partial_for(
            lax.rem(my - 1 + N_DEV, N_DEV), 0).astype(jnp.bfloat16)
        ccw_ref[N_DEV - 1] = partial_for(
            lax.rem(my + 1, N_DEV), nh).astype(jnp.bfloat16)
        for s in range(SUB):
            mk(cw_ref, cw_send, cw_recv, right, 0, s).start()
            mk(ccw_ref, ccw_send, ccw_recv, left, 0, s).start()

        for h in range(N_DEV - 1):
            c_cw = lax.rem(my - h - 2 + 2 * N_DEV, N_DEV)
            c_ccw = lax.rem(my + h + 2, N_DEV)
            p_cw = partial_for(c_cw, 0)
            p_ccw = partial_for(c_ccw, nh)

            for s in range(SUB):
                sl = slice(s * ns, (s + 1) * ns)
                for dir_ref, send_sems, recv_sems, dst, p, col0 in (
                    (cw_ref, cw_send, cw_recv, right, p_cw, 0),
                    (ccw_ref, ccw_send, ccw_recv, left, p_ccw, nh),
                ):
                    mk(dir_ref, send_sems, recv_sems, dst, h, s).wait_recv()
                    acc = dir_ref[h, :, sl].astype(jnp.float32) + p[:, sl]
                    if h < N_DEV - 2:
                        dir_ref[h, :, sl] = acc.astype(jnp.bfloat16)
                        mk(dir_ref, send_sems, recv_sems, dst, h + 1, s).start()
                    else:
                        out_ref[:, col0 + s * ns:col0 + (s + 1) * ns] = (
                            acc * jax.nn.sigmoid(acc))

        for h in range(N_DEV - 1):
            for s in range(SUB):
                mk(cw_ref, cw_send, cw_recv, right, h, s).wait_send()
                mk(ccw_ref, ccw_send, ccw_recv, left, h, s).wait_send()

    return pl.pallas_call(
        body,
        out_shape=jax.ShapeDtypeStruct((m_per, n), jnp.float32),
        in_specs=[
            pl.BlockSpec(memory_space=pltpu.VMEM),
            pl.BlockSpec(memory_space=pltpu.VMEM),
        ],
        out_specs=pl.BlockSpec(memory_space=pltpu.VMEM),
        scratch_shapes=[
            pltpu.VMEM((k_shard, n), jnp.bfloat16),
            pltpu.VMEM((N_DEV, m_per, nh), jnp.bfloat16),
            pltpu.VMEM((N_DEV, m_per, nh), jnp.bfloat16),
            pltpu.SemaphoreType.DMA((N_DEV - 1, SUB)),
            pltpu.SemaphoreType.DMA((N_DEV - 1, SUB)),
            pltpu.SemaphoreType.DMA((N_DEV - 1, SUB)),
            pltpu.SemaphoreType.DMA((N_DEV - 1, SUB)),
        ],
        compiler_params=pltpu.CompilerParams(collective_id=0),
    )(x, w_mat)
